# baseline (speedup 1.0000x reference)
import functools
import math

import jax
import jax.numpy as jnp
from jax import lax
from jax.experimental import pallas as pl
from jax.experimental.pallas import tpu as pltpu
from jax.experimental.pallas import tpu_sc as plsc

NUM_EXPERTS = 8
TOP_K = 2
CAPACITY_FACTOR = 1.25

NC = 2
NS = 16
NW = NC * NS
LANES = 16


def _router_body(T, E, CAP, CAP_PAD, x_ref, wg_ref, e_ref, p_ref, w_ref,
                 xb_ref):
    xv = x_ref[...]
    D = xv.shape[1]
    lo_u = lax.bitcast_convert_type(
        xv[:, : D // 2].astype(jnp.bfloat16), jnp.uint16).astype(jnp.uint32)
    hi_u = lax.bitcast_convert_type(
        xv[:, D // 2:].astype(jnp.bfloat16), jnp.uint16).astype(jnp.uint32)
    xb_ref[...] = lax.bitcast_convert_type(
        lo_u | lax.shift_left(hi_u, jnp.uint32(16)), jnp.int32)
    wg = wg_ref[...]
    logits = jnp.dot(xv, wg, preferred_element_type=jnp.float32)
    m = jnp.max(logits, axis=1, keepdims=True)
    ex = jnp.exp(logits - m)
    probs = ex / jnp.sum(ex, axis=1, keepdims=True)
    lane = lax.broadcasted_iota(jnp.int32, (T, E), 1)
    w0 = jnp.max(probs, axis=1, keepdims=True)
    e0 = jnp.min(jnp.where(probs == w0, lane, E), axis=1, keepdims=True)
    probs1 = jnp.where(lane == e0, -1.0, probs)
    w1 = jnp.max(probs1, axis=1, keepdims=True)
    e1 = jnp.min(jnp.where(probs1 == w1, lane, E), axis=1, keepdims=True)
    denom = w0 + w1 + 1e-8
    w0n = w0 / denom
    w1n = w1 / denom
    oh0 = (lane == e0).astype(jnp.float32)
    oh1 = (lane == e1).astype(jnp.float32)
    s = oh0 + oh1
    c = s
    k = 1
    while k < T:
        c = c + jnp.concatenate([jnp.zeros((k, E), jnp.float32), c[: T - k]], axis=0)
        k *= 2
    excl = c - s
    pos0 = jnp.sum(oh0 * excl, axis=1, keepdims=True).astype(jnp.int32)
    pos1 = jnp.sum(oh1 * excl, axis=1, keepdims=True).astype(jnp.int32)
    valid0 = pos0 < CAP
    valid1 = pos1 < CAP
    p0 = jnp.where(valid0, pos0, CAP)
    p1 = jnp.where(valid1, pos1, CAP)
    we0 = jnp.where(valid0, w0n, 0.0)
    we1 = jnp.where(valid1, w1n, 0.0)
    e_ref[...] = jnp.concatenate([e0, e1], axis=1)
    p_ref[...] = jnp.concatenate([p0, p1], axis=1)
    w_ref[...] = jnp.concatenate([we0, we1], axis=1)


def _router(xf, Wg, CAP, CAP_PAD):
    T, D = xf.shape
    E = Wg.shape[1]
    return pl.pallas_call(
        functools.partial(_router_body, T, E, CAP, CAP_PAD),
        out_shape=(
            jax.ShapeDtypeStruct((T, TOP_K), jnp.int32),
            jax.ShapeDtypeStruct((T, TOP_K), jnp.int32),
            jax.ShapeDtypeStruct((T, TOP_K), jnp.float32),
            jax.ShapeDtypeStruct((T, D // 2), jnp.int32),
        ),
    )(xf, Wg)


def _dispatch(ef, pf, wf, xb, CAP_PAD):
    NR, RL = ef.shape
    T, DW = xb.shape
    E = NUM_EXPERTS
    TOTAL = E * CAP_PAD
    ROWS = TOTAL // NW
    CHUNK = ROWS
    RPW = NR // NS
    mesh = plsc.VectorSubcoreMesh(core_axis_name="c", subcore_axis_name="s")

    @functools.partial(
        pl.kernel,
        out_type=(
            jax.ShapeDtypeStruct((TOTAL,), jnp.float32),
            jax.ShapeDtypeStruct((TOTAL, DW), jnp.int32),
        ),
        mesh=mesh,
        scratch_types=[
            pltpu.VMEM((RPW, RL), jnp.int32),
            pltpu.VMEM((RPW, RL), jnp.int32),
            pltpu.VMEM((RPW, RL), jnp.float32),
            pltpu.VMEM((RPW, RL), jnp.int32),
            pltpu.VMEM((RPW, RL), jnp.int32),
            pltpu.VMEM((ROWS,), jnp.int32),
            pltpu.VMEM((CHUNK, DW), jnp.int32),
            pltpu.VMEM_SHARED((TOTAL,), jnp.int32),
            pltpu.VMEM_SHARED((TOTAL,), jnp.float32),
            pltpu.SemaphoreType.DMA,
        ],
        compiler_params=pltpu.CompilerParams(needs_layout_passes=False),
    )
    def body(e_hbm, p_hbm, w_hbm, x_hbm, zi_hbm, zf_hbm, wt_hbm, xin_hbm,
             ev, pv, wv, tokv, slotv, myidx, bufa,
             shr_src, shr_wt, gsa):
        cid = lax.axis_index("c")
        sid = lax.axis_index("s")
        wid = sid * NC + cid

        @pl.when(sid == 0)
        def _():
            pltpu.sync_copy(zi_hbm, shr_src)
            pltpu.sync_copy(zf_hbm, shr_wt)

        pltpu.sync_copy(e_hbm.at[pl.ds(RPW * sid, RPW)], ev)
        pltpu.sync_copy(p_hbm.at[pl.ds(RPW * sid, RPW)], pv)
        pltpu.sync_copy(w_hbm.at[pl.ds(RPW * sid, RPW)], wv)
        for j in range(RPW):
            for c in range(RL // LANES):
                sl = pl.ds(c * LANES, LANES)
                e16 = ev[j, sl]
                p16 = pv[j, sl]
                base = sid * (RPW * RL) + j * RL + c * LANES
                tokv[j, sl] = lax.shift_right_logical(
                    base + lax.iota(jnp.int32, LANES), 1)
                slotv[j, sl] = e16 * CAP_PAD + p16
                ev[j, sl] = p16 * E + e16
        plsc.subcore_barrier()
        for j in range(RPW):
            pltpu.sync_copy(tokv.at[j], shr_src.at[slotv.at[j]])
            pltpu.sync_copy(wv.at[j], shr_wt.at[ev.at[j]])
        plsc.subcore_barrier()

        @pl.when(jnp.logical_and(cid == 0, sid == 0))
        def _():
            pltpu.sync_copy(shr_wt, wt_hbm)

        pltpu.sync_copy(shr_src.at[pl.ds(wid * ROWS, ROWS)], myidx)
        GC = ROWS // 3
        cps = [
            pltpu.async_copy(
                x_hbm.at[myidx.at[pl.ds(ci * GC, GC)]],
                bufa.at[pl.ds(ci * GC, GC)], gsa)
            for ci in range(3)
        ]
        for cp in cps:
            cp.wait()
        pltpu.sync_copy(bufa, xin_hbm.at[pl.ds(wid * ROWS, ROWS)])

    return body(ef, pf, wf, xb,
                jnp.arange(TOTAL, dtype=jnp.int32) % T,
                jnp.zeros((TOTAL,), jnp.float32))


def _ffn_body(xin_ref, w1_ref, b1_ref, w2_ref, b2_ref, wt_ref, out_ref):
    e_idx = pl.program_id(0)
    wu = lax.bitcast_convert_type(xin_ref[0], jnp.uint32)
    xlo = lax.bitcast_convert_type(
        lax.shift_left(wu, jnp.uint32(16)), jnp.float32).astype(jnp.bfloat16)
    xhi = lax.bitcast_convert_type(
        wu & jnp.uint32(0xFFFF0000), jnp.float32).astype(jnp.bfloat16)
    w1 = w1_ref[0].astype(jnp.bfloat16)
    DH = w1.shape[0] // 2
    h = (jnp.dot(xlo, w1[:DH], preferred_element_type=jnp.float32)
         + jnp.dot(xhi, w1[DH:], preferred_element_type=jnp.float32)
         + b1_ref[0])
    h = jnp.maximum(h, 0.0).astype(jnp.bfloat16)
    w2 = w2_ref[0].astype(jnp.bfloat16)
    part = jnp.dot(h, w2, preferred_element_type=jnp.float32)
    E = wt_ref.shape[1]
    oh = (lax.broadcasted_iota(jnp.int32, (E, 1), 0) == e_idx).astype(jnp.float32)
    wcol = jnp.dot(wt_ref[...], oh, preferred_element_type=jnp.float32)
    out_ref[0] = (part + b2_ref[0]) * wcol


def _ffn(xin, W1, b1, W2, b2, wt):
    E, CAP_PAD, DW = xin.shape
    D = W1.shape[1]
    H = W1.shape[2]
    grid = (E,)
    return pl.pallas_call(
        _ffn_body,
        grid=grid,
        in_specs=[
            pl.BlockSpec((1, CAP_PAD, DW), lambda e: (e, 0, 0)),
            pl.BlockSpec((1, D, H), lambda e: (e, 0, 0)),
            pl.BlockSpec((1, 1, H), lambda e: (e, 0, 0)),
            pl.BlockSpec((1, H, D), lambda e: (e, 0, 0)),
            pl.BlockSpec((1, 1, D), lambda e: (e, 0, 0)),
            pl.BlockSpec((CAP_PAD, E), lambda e: (0, 0)),
        ],
        out_specs=pl.BlockSpec((1, CAP_PAD, D), lambda e: (e, 0, 0)),
        out_shape=jax.ShapeDtypeStruct((E, CAP_PAD, D), jnp.float32),
        compiler_params=pltpu.CompilerParams(
            dimension_semantics=("arbitrary",),
        ),
    )(xin, W1, b1.reshape(E, 1, H), W2, b2.reshape(E, 1, D), wt)


def _combine(e2d, p2d, yw, T, CAP_PAD):
    NR, RL = e2d.shape
    D = yw.shape[1]
    TPW = T // NW
    TCHUNK = 16
    NCH = TPW // TCHUNK
    mesh = plsc.VectorSubcoreMesh(core_axis_name="c", subcore_axis_name="s")

    @functools.partial(
        pl.kernel,
        out_type=jax.ShapeDtypeStruct((T, D), jnp.float32),
        mesh=mesh,
        scratch_types=[
            pltpu.VMEM((RL,), jnp.int32),
            pltpu.VMEM((RL,), jnp.int32),
            pltpu.VMEM((TPW,), jnp.int32),
            pltpu.VMEM((TPW,), jnp.int32),
            pltpu.VMEM((TCHUNK, D), jnp.float32),
            pltpu.VMEM((TCHUNK, D), jnp.float32),
            pltpu.VMEM((TCHUNK, D), jnp.float32),
            pltpu.VMEM((TCHUNK, D), jnp.float32),
            pltpu.SemaphoreType.DMA,
            pltpu.SemaphoreType.DMA,
            pltpu.SemaphoreType.DMA,
            pltpu.SemaphoreType.DMA,
            pltpu.SemaphoreType.DMA,
            pltpu.SemaphoreType.DMA,
        ],
        compiler_params=pltpu.CompilerParams(needs_layout_passes=False),
    )
    def body(e_hbm, p_hbm, yw_hbm, out_hbm,
             ev, pv, idx0_v, idx1_v, b0a, b1a, b0b, b1b,
             g0a, g1a, g0b, g1b, osa, osb):
        cid = lax.axis_index("c")
        sid = lax.axis_index("s")
        wid = sid * NC + cid
        pltpu.sync_copy(e_hbm.at[wid], ev)
        pltpu.sync_copy(p_hbm.at[wid], pv)
        for c2 in range(TPW // LANES):
            l0 = 2 * (c2 * LANES + lax.iota(jnp.int32, LANES))
            l1 = l0 + 1
            s0 = (plsc.load_gather(ev, [l0]) * CAP_PAD
                  + plsc.load_gather(pv, [l0]))
            s1 = (plsc.load_gather(ev, [l1]) * CAP_PAD
                  + plsc.load_gather(pv, [l1]))
            idx0_v[pl.ds(c2 * LANES, LANES)] = s0
            idx1_v[pl.ds(c2 * LANES, LANES)] = s1
        b0 = (b0a, b0b)
        b1 = (b1a, b1b)
        g0 = (g0a, g0b)
        g1 = (g1a, g1b)
        osem = (osa, osb)
        gathers = [None] * NCH
        outs = [None, None]

        def start(ci):
            s = ci % 2
            gathers[ci] = (
                pltpu.async_copy(
                    yw_hbm.at[idx0_v.at[pl.ds(ci * TCHUNK, TCHUNK)]],
                    b0[s], g0[s]),
                pltpu.async_copy(
                    yw_hbm.at[idx1_v.at[pl.ds(ci * TCHUNK, TCHUNK)]],
                    b1[s], g1[s]))

        start(0)
        start(1)
        for ci in range(NCH):
            s = ci % 2
            cp0, cp1 = gathers[ci]
            cp0.wait()
            cp1.wait()
            for r in range(TCHUNK):
                def aloop(c, carry, r=r, s=s):
                    col = c * (4 * LANES)
                    for u in range(4):
                        off = col + u * LANES
                        b0[s][r, pl.ds(off, LANES)] = (
                            b0[s][r, pl.ds(off, LANES)]
                            + b1[s][r, pl.ds(off, LANES)])
                    return carry
                lax.fori_loop(0, D // (4 * LANES), aloop, 0)
            outs[s] = pltpu.async_copy(
                b0[s], out_hbm.at[pl.ds(wid * TPW + ci * TCHUNK, TCHUNK)],
                osem[s])
            if ci + 2 < NCH:
                outs[s].wait()
                outs[s] = None
                start(ci + 2)
        for cp in outs:
            if cp is not None:
                cp.wait()

    return body(e2d, p2d, yw)


def kernel(x, Wg, W1, b1, W2, b2):
    B, S, D = x.shape
    T = B * S
    E = Wg.shape[1]
    NI = T * TOP_K
    CAP = int(math.ceil(NI / E * CAPACITY_FACTOR))
    CAP_PAD = CAP + 32

    xf = x.reshape(T, D)
    e2, p2, w2, xb = _router(xf, Wg, CAP, CAP_PAD)
    e2d = e2.reshape(NI // 128, 128)
    p2d = p2.reshape(NI // 128, 128)
    w2d = w2.reshape(NI // 128, 128)
    wt, xin = _dispatch(e2d, p2d, w2d, xb, CAP_PAD)
    yw = _ffn(xin.reshape(E, CAP_PAD, D // 2), W1, b1, W2, b2,
              wt.reshape(CAP_PAD, E))
    out = _combine(e2d, p2d, yw.reshape(E * CAP_PAD, D), T, CAP_PAD)
    return out.reshape(B, S, D)


# speedup vs baseline: 1.0324x; 1.0324x over previous
import functools
import math

import jax
import jax.numpy as jnp
from jax import lax
from jax.experimental import pallas as pl
from jax.experimental.pallas import tpu as pltpu
from jax.experimental.pallas import tpu_sc as plsc

NUM_EXPERTS = 8
TOP_K = 2
CAPACITY_FACTOR = 1.25

NC = 2
NS = 16
NW = NC * NS
LANES = 16


def _router_body(T, E, CAP, CAP_PAD, x_ref, wg_ref, e_ref, p_ref, w_ref,
                 xb_ref):
    xv = x_ref[...]
    D = xv.shape[1]
    lo_u = lax.bitcast_convert_type(
        xv[:, : D // 2].astype(jnp.bfloat16), jnp.uint16).astype(jnp.uint32)
    hi_u = lax.bitcast_convert_type(
        xv[:, D // 2:].astype(jnp.bfloat16), jnp.uint16).astype(jnp.uint32)
    xb_ref[...] = lax.bitcast_convert_type(
        lo_u | lax.shift_left(hi_u, jnp.uint32(16)), jnp.int32)
    wg = wg_ref[...]
    logits = jnp.dot(xv, wg, preferred_element_type=jnp.float32)
    m = jnp.max(logits, axis=1, keepdims=True)
    ex = jnp.exp(logits - m)
    probs = ex / jnp.sum(ex, axis=1, keepdims=True)
    lane = lax.broadcasted_iota(jnp.int32, (T, E), 1)
    w0 = jnp.max(probs, axis=1, keepdims=True)
    e0 = jnp.min(jnp.where(probs == w0, lane, E), axis=1, keepdims=True)
    probs1 = jnp.where(lane == e0, -1.0, probs)
    w1 = jnp.max(probs1, axis=1, keepdims=True)
    e1 = jnp.min(jnp.where(probs1 == w1, lane, E), axis=1, keepdims=True)
    denom = w0 + w1 + 1e-8
    w0n = w0 / denom
    w1n = w1 / denom
    oh0 = (lane == e0).astype(jnp.float32)
    oh1 = (lane == e1).astype(jnp.float32)
    s = oh0 + oh1
    c = s
    k = 1
    while k < T:
        c = c + jnp.concatenate([jnp.zeros((k, E), jnp.float32), c[: T - k]], axis=0)
        k *= 2
    excl = c - s
    pos0 = jnp.sum(oh0 * excl, axis=1, keepdims=True).astype(jnp.int32)
    pos1 = jnp.sum(oh1 * excl, axis=1, keepdims=True).astype(jnp.int32)
    valid0 = pos0 < CAP
    valid1 = pos1 < CAP
    p0 = jnp.where(valid0, pos0, CAP)
    p1 = jnp.where(valid1, pos1, CAP)
    we0 = jnp.where(valid0, w0n, 0.0)
    we1 = jnp.where(valid1, w1n, 0.0)
    e_ref[...] = jnp.concatenate([e0, e1], axis=1)
    p_ref[...] = jnp.concatenate([p0, p1], axis=1)
    w_ref[...] = jnp.concatenate([we0, we1], axis=1)


def _router(xf, Wg, CAP, CAP_PAD):
    T, D = xf.shape
    E = Wg.shape[1]
    return pl.pallas_call(
        functools.partial(_router_body, T, E, CAP, CAP_PAD),
        out_shape=(
            jax.ShapeDtypeStruct((T, TOP_K), jnp.int32),
            jax.ShapeDtypeStruct((T, TOP_K), jnp.int32),
            jax.ShapeDtypeStruct((T, TOP_K), jnp.float32),
            jax.ShapeDtypeStruct((T, D // 2), jnp.int32),
        ),
    )(xf, Wg)


def _dispatch(ef, pf, wf, xb, CAP_PAD):
    NR, RL = ef.shape
    T, DW = xb.shape
    E = NUM_EXPERTS
    TOTAL = E * CAP_PAD
    ROWS = TOTAL // NW
    CHUNK = ROWS
    RPW = NR // NS
    mesh = plsc.VectorSubcoreMesh(core_axis_name="c", subcore_axis_name="s")

    @functools.partial(
        pl.kernel,
        out_type=(
            jax.ShapeDtypeStruct((TOTAL,), jnp.float32),
            jax.ShapeDtypeStruct((TOTAL, DW), jnp.int32),
        ),
        mesh=mesh,
        scratch_types=[
            pltpu.VMEM((RPW, RL), jnp.int32),
            pltpu.VMEM((RPW, RL), jnp.int32),
            pltpu.VMEM((RPW, RL), jnp.float32),
            pltpu.VMEM((RPW, RL), jnp.int32),
            pltpu.VMEM((RPW, RL), jnp.int32),
            pltpu.VMEM((ROWS,), jnp.int32),
            pltpu.VMEM((CHUNK, DW), jnp.int32),
            pltpu.VMEM_SHARED((TOTAL,), jnp.int32),
            pltpu.VMEM_SHARED((TOTAL,), jnp.float32),
            pltpu.SemaphoreType.DMA,
        ],
        compiler_params=pltpu.CompilerParams(needs_layout_passes=False),
    )
    def body(e_hbm, p_hbm, w_hbm, x_hbm, zi_hbm, zf_hbm, wt_hbm, xin_hbm,
             ev, pv, wv, tokv, slotv, myidx, bufa,
             shr_src, shr_wt, gsa):
        cid = lax.axis_index("c")
        sid = lax.axis_index("s")
        wid = sid * NC + cid

        @pl.when(sid == 0)
        def _():
            pltpu.sync_copy(zi_hbm, shr_src)
            pltpu.sync_copy(zf_hbm, shr_wt)

        pltpu.sync_copy(e_hbm.at[pl.ds(RPW * sid, RPW)], ev)
        pltpu.sync_copy(p_hbm.at[pl.ds(RPW * sid, RPW)], pv)
        pltpu.sync_copy(w_hbm.at[pl.ds(RPW * sid, RPW)], wv)
        for j in range(RPW):
            for c in range(RL // LANES):
                sl = pl.ds(c * LANES, LANES)
                e16 = ev[j, sl]
                p16 = pv[j, sl]
                base = sid * (RPW * RL) + j * RL + c * LANES
                tokv[j, sl] = lax.shift_right_logical(
                    base + lax.iota(jnp.int32, LANES), 1)
                slotv[j, sl] = e16 * CAP_PAD + p16
                ev[j, sl] = p16 * E + e16
        plsc.subcore_barrier()
        for j in range(RPW):
            pltpu.sync_copy(tokv.at[j], shr_src.at[slotv.at[j]])
            pltpu.sync_copy(wv.at[j], shr_wt.at[ev.at[j]])
        plsc.subcore_barrier()

        @pl.when(jnp.logical_and(cid == 0, sid == 0))
        def _():
            pltpu.sync_copy(shr_wt, wt_hbm)

        pltpu.sync_copy(shr_src.at[pl.ds(wid * ROWS, ROWS)], myidx)
        GC = ROWS // 3
        cps = [
            pltpu.async_copy(
                x_hbm.at[myidx.at[pl.ds(ci * GC, GC)]],
                bufa.at[pl.ds(ci * GC, GC)], gsa)
            for ci in range(3)
        ]
        for cp in cps:
            cp.wait()
        pltpu.sync_copy(bufa, xin_hbm.at[pl.ds(wid * ROWS, ROWS)])

    return body(ef, pf, wf, xb,
                jnp.arange(TOTAL, dtype=jnp.int32) % T,
                jnp.zeros((TOTAL,), jnp.float32))


def _ffn_body(xin_ref, w1_ref, b1_ref, w2_ref, b2_ref, wt_ref, out_ref):
    e_idx = pl.program_id(0)
    wu = lax.bitcast_convert_type(xin_ref[0], jnp.uint32)
    xlo = lax.bitcast_convert_type(
        lax.shift_left(wu, jnp.uint32(16)), jnp.float32).astype(jnp.bfloat16)
    xhi = lax.bitcast_convert_type(
        wu & jnp.uint32(0xFFFF0000), jnp.float32).astype(jnp.bfloat16)
    w1 = w1_ref[0].astype(jnp.bfloat16)
    DH = w1.shape[0] // 2
    h = (jnp.dot(xlo, w1[:DH], preferred_element_type=jnp.float32)
         + jnp.dot(xhi, w1[DH:], preferred_element_type=jnp.float32)
         + b1_ref[0])
    h = jnp.maximum(h, 0.0).astype(jnp.bfloat16)
    w2 = w2_ref[0].astype(jnp.bfloat16)
    part = jnp.dot(h, w2, preferred_element_type=jnp.float32)
    E = wt_ref.shape[1]
    oh = (lax.broadcasted_iota(jnp.int32, (E, 1), 0) == e_idx).astype(jnp.float32)
    wcol = jnp.dot(wt_ref[...], oh, preferred_element_type=jnp.float32)
    y = (part + b2_ref[0]) * wcol
    DHW = y.shape[1] // 2
    ylo = lax.bitcast_convert_type(
        y[:, :DHW].astype(jnp.bfloat16), jnp.uint16).astype(jnp.uint32)
    yhi = lax.bitcast_convert_type(
        y[:, DHW:].astype(jnp.bfloat16), jnp.uint16).astype(jnp.uint32)
    out_ref[0] = lax.bitcast_convert_type(
        ylo | lax.shift_left(yhi, jnp.uint32(16)), jnp.int32)


def _ffn(xin, W1, b1, W2, b2, wt):
    E, CAP_PAD, DW = xin.shape
    D = W1.shape[1]
    H = W1.shape[2]
    grid = (E,)
    return pl.pallas_call(
        _ffn_body,
        grid=grid,
        in_specs=[
            pl.BlockSpec((1, CAP_PAD, DW), lambda e: (e, 0, 0)),
            pl.BlockSpec((1, D, H), lambda e: (e, 0, 0)),
            pl.BlockSpec((1, 1, H), lambda e: (e, 0, 0)),
            pl.BlockSpec((1, H, D), lambda e: (e, 0, 0)),
            pl.BlockSpec((1, 1, D), lambda e: (e, 0, 0)),
            pl.BlockSpec((CAP_PAD, E), lambda e: (0, 0)),
        ],
        out_specs=pl.BlockSpec((1, CAP_PAD, D // 2), lambda e: (e, 0, 0)),
        out_shape=jax.ShapeDtypeStruct((E, CAP_PAD, D // 2), jnp.int32),
        compiler_params=pltpu.CompilerParams(
            dimension_semantics=("arbitrary",),
        ),
    )(xin, W1, b1.reshape(E, 1, H), W2, b2.reshape(E, 1, D), wt)


def _combine(e2d, p2d, yw, T, CAP_PAD):
    NR, RL = e2d.shape
    DW = yw.shape[1]
    D = DW * 2
    TPW = T // NW
    TCHUNK = 16
    NCH = TPW // TCHUNK
    mesh = plsc.VectorSubcoreMesh(core_axis_name="c", subcore_axis_name="s")

    @functools.partial(
        pl.kernel,
        out_type=jax.ShapeDtypeStruct((T, D), jnp.float32),
        mesh=mesh,
        scratch_types=[
            pltpu.VMEM((RL,), jnp.int32),
            pltpu.VMEM((RL,), jnp.int32),
            pltpu.VMEM((TPW,), jnp.int32),
            pltpu.VMEM((TPW,), jnp.int32),
            pltpu.VMEM((TCHUNK, DW), jnp.int32),
            pltpu.VMEM((TCHUNK, DW), jnp.int32),
            pltpu.VMEM((TCHUNK, DW), jnp.int32),
            pltpu.VMEM((TCHUNK, DW), jnp.int32),
            pltpu.VMEM((TCHUNK, D), jnp.float32),
            pltpu.VMEM((TCHUNK, D), jnp.float32),
            pltpu.SemaphoreType.DMA,
            pltpu.SemaphoreType.DMA,
            pltpu.SemaphoreType.DMA,
            pltpu.SemaphoreType.DMA,
            pltpu.SemaphoreType.DMA,
            pltpu.SemaphoreType.DMA,
        ],
        compiler_params=pltpu.CompilerParams(needs_layout_passes=False),
    )
    def body(e_hbm, p_hbm, yw_hbm, out_hbm,
             ev, pv, idx0_v, idx1_v, b0a, b1a, b0b, b1b, oba, obb,
             g0a, g1a, g0b, g1b, osa, osb):
        cid = lax.axis_index("c")
        sid = lax.axis_index("s")
        wid = sid * NC + cid
        pltpu.sync_copy(e_hbm.at[wid], ev)
        pltpu.sync_copy(p_hbm.at[wid], pv)
        for c2 in range(TPW // LANES):
            l0 = 2 * (c2 * LANES + lax.iota(jnp.int32, LANES))
            l1 = l0 + 1
            s0 = (plsc.load_gather(ev, [l0]) * CAP_PAD
                  + plsc.load_gather(pv, [l0]))
            s1 = (plsc.load_gather(ev, [l1]) * CAP_PAD
                  + plsc.load_gather(pv, [l1]))
            idx0_v[pl.ds(c2 * LANES, LANES)] = s0
            idx1_v[pl.ds(c2 * LANES, LANES)] = s1
        b0 = (b0a, b0b)
        b1 = (b1a, b1b)
        ob = (oba, obb)
        g0 = (g0a, g0b)
        g1 = (g1a, g1b)
        osem = (osa, osb)
        gathers = [None] * NCH
        outs = [None, None]

        def start(ci):
            s = ci % 2
            gathers[ci] = (
                pltpu.async_copy(
                    yw_hbm.at[idx0_v.at[pl.ds(ci * TCHUNK, TCHUNK)]],
                    b0[s], g0[s]),
                pltpu.async_copy(
                    yw_hbm.at[idx1_v.at[pl.ds(ci * TCHUNK, TCHUNK)]],
                    b1[s], g1[s]))

        start(0)
        start(1)
        hi_mask = jnp.full((LANES,), 0xFFFF0000, jnp.uint32)
        for ci in range(NCH):
            s = ci % 2
            cp0, cp1 = gathers[ci]
            cp0.wait()
            cp1.wait()
            for r in range(TCHUNK):
                def aloop(c, carry, r=r, s=s):
                    for u in range(2):
                        off = (2 * c + u) * LANES
                        w0 = lax.bitcast_convert_type(
                            b0[s][r, pl.ds(off, LANES)], jnp.uint32)
                        w1 = lax.bitcast_convert_type(
                            b1[s][r, pl.ds(off, LANES)], jnp.uint32)
                        lo = (lax.bitcast_convert_type(
                                  lax.shift_left(w0, jnp.uint32(16)),
                                  jnp.float32)
                              + lax.bitcast_convert_type(
                                  lax.shift_left(w1, jnp.uint32(16)),
                                  jnp.float32))
                        hi = (lax.bitcast_convert_type(w0 & hi_mask,
                                                       jnp.float32)
                              + lax.bitcast_convert_type(w1 & hi_mask,
                                                         jnp.float32))
                        ob[s][r, pl.ds(off, LANES)] = lo
                        ob[s][r, pl.ds(DW + off, LANES)] = hi
                    return carry
                lax.fori_loop(0, DW // (2 * LANES), aloop, 0)
            outs[s] = pltpu.async_copy(
                ob[s], out_hbm.at[pl.ds(wid * TPW + ci * TCHUNK, TCHUNK)],
                osem[s])
            if ci + 2 < NCH:
                outs[s].wait()
                outs[s] = None
                start(ci + 2)
        for cp in outs:
            if cp is not None:
                cp.wait()

    return body(e2d, p2d, yw)


def kernel(x, Wg, W1, b1, W2, b2):
    B, S, D = x.shape
    T = B * S
    E = Wg.shape[1]
    NI = T * TOP_K
    CAP = int(math.ceil(NI / E * CAPACITY_FACTOR))
    CAP_PAD = CAP + 32

    xf = x.reshape(T, D)
    e2, p2, w2, xb = _router(xf, Wg, CAP, CAP_PAD)
    e2d = e2.reshape(NI // 128, 128)
    p2d = p2.reshape(NI // 128, 128)
    w2d = w2.reshape(NI // 128, 128)
    wt, xin = _dispatch(e2d, p2d, w2d, xb, CAP_PAD)
    yw = _ffn(xin.reshape(E, CAP_PAD, D // 2), W1, b1, W2, b2,
              wt.reshape(CAP_PAD, E))
    out = _combine(e2d, p2d, yw.reshape(E * CAP_PAD, D // 2), T, CAP_PAD)
    return out.reshape(B, S, D)
